# Initial kernel scaffold; baseline (speedup 1.0000x reference)
import functools

import jax
import jax.numpy as jnp
from jax import lax
from jax.experimental import pallas as pl
from jax.experimental.pallas import tpu as pltpu
from jax.experimental.pallas import tpu_sc as plsc

N_NODES = 10000
N_EDGES = 320000
NUM_COUNTS = 16
NUM_FEATS = 128
HALF = NUM_FEATS // 2
LANES = 16

NUM_CORES = 2
NUM_SUBCORES = 16
EDGES_PER_TILE = N_EDGES // NUM_SUBCORES
CHUNK = 80
NUM_CHUNKS = EDGES_PER_TILE // CHUNK
NODES_PER_TILE = N_NODES // NUM_SUBCORES

TC_BLOCK = 4000


def _score_body(cnt_ref, imp_ref, out_ref):
    imp = imp_ref[...]
    m = jnp.max(imp, axis=0, keepdims=True)
    e = jnp.exp(imp - m)
    w = e / jnp.sum(e, axis=0, keepdims=True)
    s = jnp.dot(cnt_ref[...], w, preferred_element_type=jnp.float32)
    out_ref[0] = s[:, :HALF]
    out_ref[1] = s[:, HALF:]


def _edge_scores(cnt, importance):
    return pl.pallas_call(
        _score_body,
        grid=(N_EDGES // TC_BLOCK,),
        in_specs=[
            pl.BlockSpec((TC_BLOCK, NUM_COUNTS), lambda i: (i, 0)),
            pl.BlockSpec((NUM_COUNTS, NUM_FEATS), lambda i: (0, 0)),
        ],
        out_specs=pl.BlockSpec((2, TC_BLOCK, HALF), lambda i: (0, i, 0)),
        out_shape=jax.ShapeDtypeStruct((2, N_EDGES, HALF), jnp.float32),
    )(cnt, importance)


def _sc_body(emb_ref, src_ref, dst_ref, score_ref, out_ref,
             acc_e, acc_s, idx_v, dst_v, score_v, rows_v, resa_v, resb_v, sem):
    c = lax.axis_index("c")
    s = lax.axis_index("s")
    node0 = s * NODES_PER_TILE

    def zero_row(n, carry):
        for b in range(HALF // LANES):
            resa_v[n, pl.ds(b * LANES, LANES)] = jnp.zeros((LANES,), jnp.float32)
        return carry
    lax.fori_loop(0, NODES_PER_TILE, zero_row, 0)
    pltpu.sync_copy(resa_v, acc_e.at[pl.ds(node0, NODES_PER_TILE)])
    pltpu.sync_copy(resa_v, acc_s.at[pl.ds(node0, NODES_PER_TILE)])
    plsc.subcore_barrier()

    row_off = c * N_NODES
    score_off = c * N_EDGES

    def chunk(i, carry):
        base = s * EDGES_PER_TILE + i * CHUNK
        pltpu.sync_copy(src_ref.at[pl.ds(base, CHUNK)], idx_v)
        pltpu.sync_copy(dst_ref.at[pl.ds(base, CHUNK)], dst_v)
        pltpu.sync_copy(score_ref.at[pl.ds(score_off + base, CHUNK)], score_v)
        for j in range(CHUNK // LANES):
            sl = pl.ds(j * LANES, LANES)
            idx_v[sl] = idx_v[sl] + row_off
        pltpu.async_copy(emb_ref.at[idx_v], rows_v, sem).wait()

        def emul(e, inner):
            for b in range(HALF // LANES):
                sl = pl.ds(b * LANES, LANES)
                rows_v[e, sl] = rows_v[e, sl] * score_v[e, sl]
            return inner
        lax.fori_loop(0, CHUNK, emul, 0)

        pltpu.sync_copy(rows_v, acc_e.at[dst_v], add=True)
        pltpu.sync_copy(score_v, acc_s.at[dst_v], add=True)
        return carry
    lax.fori_loop(0, NUM_CHUNKS, chunk, 0)
    plsc.subcore_barrier()

    pltpu.sync_copy(acc_e.at[pl.ds(node0, NODES_PER_TILE)], resa_v)
    pltpu.sync_copy(acc_s.at[pl.ds(node0, NODES_PER_TILE)], resb_v)

    def ndiv(n, carry):
        for b in range(HALF // LANES):
            sl = pl.ds(b * LANES, LANES)
            resa_v[n, sl] = resa_v[n, sl] / resb_v[n, sl]
        return carry
    lax.fori_loop(0, NODES_PER_TILE, ndiv, 0)
    pltpu.sync_copy(resa_v, out_ref.at[pl.ds(c * N_NODES + node0, NODES_PER_TILE)])


@functools.partial(
    pl.kernel,
    out_type=jax.ShapeDtypeStruct((2 * N_NODES, HALF), jnp.float32),
    mesh=plsc.VectorSubcoreMesh(
        core_axis_name="c", subcore_axis_name="s",
        num_cores=NUM_CORES, num_subcores=NUM_SUBCORES),
    scratch_types=[
        pltpu.VMEM_SHARED((N_NODES, HALF), jnp.float32),
        pltpu.VMEM_SHARED((N_NODES, HALF), jnp.float32),
        pltpu.VMEM((CHUNK,), jnp.int32),
        pltpu.VMEM((CHUNK,), jnp.int32),
        pltpu.VMEM((CHUNK, HALF), jnp.float32),
        pltpu.VMEM((CHUNK, HALF), jnp.float32),
        pltpu.VMEM((NODES_PER_TILE, HALF), jnp.float32),
        pltpu.VMEM((NODES_PER_TILE, HALF), jnp.float32),
        pltpu.SemaphoreType.DMA,
    ],
)
def _sc_aggregate(emb_ref, src_ref, dst_ref, score_ref, out_ref, *scratch):
    _sc_body(emb_ref, src_ref, dst_ref, score_ref, out_ref, *scratch)


def kernel(embedding, edge_index, cnt, importance):
    src = edge_index[0].astype(jnp.int32)
    dst = edge_index[1].astype(jnp.int32)
    score_both = _edge_scores(cnt, importance).reshape(2 * N_EDGES, HALF)
    emb_both = jnp.concatenate([embedding[:, :HALF], embedding[:, HALF:]], axis=0)
    out_both = _sc_aggregate(emb_both, src, dst, score_both)
    return jnp.concatenate([out_both[:N_NODES], out_both[N_NODES:]], axis=1)


# speedup vs baseline: 1.9844x; 1.9844x over previous
import functools

import jax
import jax.numpy as jnp
from jax import lax
from jax.experimental import pallas as pl
from jax.experimental.pallas import tpu as pltpu
from jax.experimental.pallas import tpu_sc as plsc

N_NODES = 10000
N_EDGES = 320000
NUM_COUNTS = 16
NUM_FEATS = 128
HALF = NUM_FEATS // 2
LANES = 16

NUM_CORES = 2
NUM_SUBCORES = 16
EDGES_PER_TILE = N_EDGES // NUM_SUBCORES
CHUNK = 80
NUM_CHUNKS = EDGES_PER_TILE // CHUNK
N_PAD = 10240
NODES_PER_TILE = N_PAD // NUM_SUBCORES
NP_CHUNK = 128
NP_STEPS = NODES_PER_TILE // NP_CHUNK

TC_BLOCK = 4000


def _score_body(cnt_ref, imp_ref, out_ref):
    imp = imp_ref[...]
    m = jnp.max(imp, axis=0, keepdims=True)
    e = jnp.exp(imp - m)
    w = e / jnp.sum(e, axis=0, keepdims=True)
    s = jnp.dot(cnt_ref[...], w, preferred_element_type=jnp.float32)
    out_ref[0] = s[:, :HALF]
    out_ref[1] = s[:, HALF:]


def _edge_scores(cnt, importance):
    return pl.pallas_call(
        _score_body,
        grid=(N_EDGES // TC_BLOCK,),
        in_specs=[
            pl.BlockSpec((TC_BLOCK, NUM_COUNTS), lambda i: (i, 0)),
            pl.BlockSpec((NUM_COUNTS, NUM_FEATS), lambda i: (0, 0)),
        ],
        out_specs=pl.BlockSpec((2, TC_BLOCK, HALF), lambda i: (0, i, 0)),
        out_shape=jax.ShapeDtypeStruct((2, N_EDGES, HALF), jnp.float32),
    )(cnt, importance)


def _sc_body(emb_ref, src_ref, dst_ref, score_ref, out_ref,
             acc_e, acc_s, idx_v, dst_v, score_v, rows_v, resa_v, resb_v, sem):
    c = lax.axis_index("c")
    s = lax.axis_index("s")
    node0 = s * NODES_PER_TILE

    def zero_row(n, carry):
        for b in range(HALF // LANES):
            resa_v[n, pl.ds(b * LANES, LANES)] = jnp.zeros((LANES,), jnp.float32)
        return carry
    lax.fori_loop(0, NP_CHUNK, zero_row, 0)

    def zero_chunk(k, carry):
        r0 = node0 + k * NP_CHUNK
        pltpu.sync_copy(resa_v, acc_e.at[pl.ds(r0, NP_CHUNK)])
        pltpu.sync_copy(resa_v, acc_s.at[pl.ds(r0, NP_CHUNK)])
        return carry
    lax.fori_loop(0, NP_STEPS, zero_chunk, 0)
    plsc.subcore_barrier()

    row_off = c * N_NODES
    score_off = c * N_EDGES

    def chunk(i, carry):
        base = s * EDGES_PER_TILE + i * CHUNK
        pltpu.sync_copy(src_ref.at[pl.ds(base, CHUNK)], idx_v)
        pltpu.sync_copy(dst_ref.at[pl.ds(base, CHUNK)], dst_v)
        pltpu.sync_copy(score_ref.at[pl.ds(score_off + base, CHUNK)], score_v)
        for j in range(CHUNK // LANES):
            sl = pl.ds(j * LANES, LANES)
            idx_v[sl] = idx_v[sl] + row_off
        pltpu.async_copy(emb_ref.at[idx_v], rows_v, sem).wait()

        def emul(e, inner):
            for b in range(HALF // LANES):
                sl = pl.ds(b * LANES, LANES)
                rows_v[e, sl] = rows_v[e, sl] * score_v[e, sl]
            return inner
        lax.fori_loop(0, CHUNK, emul, 0)

        pltpu.sync_copy(rows_v, acc_e.at[dst_v], add=True)
        pltpu.sync_copy(score_v, acc_s.at[dst_v], add=True)
        return carry
    lax.fori_loop(0, NUM_CHUNKS, chunk, 0)
    plsc.subcore_barrier()

    def out_chunk(k, carry):
        r0 = node0 + k * NP_CHUNK
        pltpu.sync_copy(acc_e.at[pl.ds(r0, NP_CHUNK)], resa_v)
        pltpu.sync_copy(acc_s.at[pl.ds(r0, NP_CHUNK)], resb_v)

        def ndiv(n, inner):
            for b in range(HALF // LANES):
                sl = pl.ds(b * LANES, LANES)
                resa_v[n, sl] = resa_v[n, sl] / resb_v[n, sl]
            return inner
        lax.fori_loop(0, NP_CHUNK, ndiv, 0)
        pltpu.sync_copy(resa_v, out_ref.at[pl.ds(c * N_PAD + r0, NP_CHUNK)])
        return carry
    lax.fori_loop(0, NP_STEPS, out_chunk, 0)


@functools.partial(
    pl.kernel,
    out_type=jax.ShapeDtypeStruct((2 * N_PAD, HALF), jnp.float32),
    mesh=plsc.VectorSubcoreMesh(
        core_axis_name="c", subcore_axis_name="s",
        num_cores=NUM_CORES, num_subcores=NUM_SUBCORES),
    scratch_types=[
        pltpu.VMEM_SHARED((N_PAD, HALF), jnp.float32),
        pltpu.VMEM_SHARED((N_PAD, HALF), jnp.float32),
        pltpu.VMEM((CHUNK,), jnp.int32),
        pltpu.VMEM((CHUNK,), jnp.int32),
        pltpu.VMEM((CHUNK, HALF), jnp.float32),
        pltpu.VMEM((CHUNK, HALF), jnp.float32),
        pltpu.VMEM((NP_CHUNK, HALF), jnp.float32),
        pltpu.VMEM((NP_CHUNK, HALF), jnp.float32),
        pltpu.SemaphoreType.DMA,
    ],
    compiler_params=pltpu.CompilerParams(use_tc_tiling_on_sc=False),
)
def _sc_aggregate(emb_ref, src_ref, dst_ref, score_ref, out_ref, *scratch):
    _sc_body(emb_ref, src_ref, dst_ref, score_ref, out_ref, *scratch)


def kernel(embedding, edge_index, cnt, importance):
    src = edge_index[0].astype(jnp.int32)
    dst = edge_index[1].astype(jnp.int32)
    score_both = _edge_scores(cnt, importance).reshape(2 * N_EDGES, HALF)
    emb_both = jnp.concatenate([embedding[:, :HALF], embedding[:, HALF:]], axis=0)
    out_both = _sc_aggregate(emb_both, src, dst, score_both)
    return jnp.concatenate(
        [out_both[:N_NODES], out_both[N_PAD:N_PAD + N_NODES]], axis=1)


# speedup vs baseline: 2.8855x; 1.4541x over previous
import functools

import jax
import jax.numpy as jnp
from jax import lax
from jax.experimental import pallas as pl
from jax.experimental.pallas import tpu as pltpu
from jax.experimental.pallas import tpu_sc as plsc

N_NODES = 10000
N_EDGES = 320000
NUM_COUNTS = 16
NUM_FEATS = 128
LANES = 16

NUM_CORES = 2
NUM_SUBCORES = 16
EDGES_PER_CORE = N_EDGES // NUM_CORES
EDGES_PER_TILE = EDGES_PER_CORE // NUM_SUBCORES
CHUNK = 80
NUM_CHUNKS = EDGES_PER_TILE // CHUNK
N_PAD = 10240
NODES_PER_TILE = N_PAD // NUM_SUBCORES
NP_CHUNK = 64
NP_STEPS = NODES_PER_TILE // NP_CHUNK

TC_BLOCK = 4000
FIN_BLOCK = 2048


def _score_body(cnt_ref, imp_ref, out_ref):
    imp = imp_ref[...]
    m = jnp.max(imp, axis=0, keepdims=True)
    e = jnp.exp(imp - m)
    w = e / jnp.sum(e, axis=0, keepdims=True)
    out_ref[...] = jnp.dot(cnt_ref[...], w, preferred_element_type=jnp.float32)


def _edge_scores(cnt, importance):
    return pl.pallas_call(
        _score_body,
        grid=(N_EDGES // TC_BLOCK,),
        in_specs=[
            pl.BlockSpec((TC_BLOCK, NUM_COUNTS), lambda i: (i, 0)),
            pl.BlockSpec((NUM_COUNTS, NUM_FEATS), lambda i: (0, 0)),
        ],
        out_specs=pl.BlockSpec((TC_BLOCK, NUM_FEATS), lambda i: (i, 0)),
        out_shape=jax.ShapeDtypeStruct((N_EDGES, NUM_FEATS), jnp.float32),
    )(cnt, importance)


def _sc_body(emb_ref, src_ref, dst_ref, score_ref, cnt_ref,
             sums_ref, csums_ref,
             acc_e, acc_c, idx_v, dst_v, score_v, rows_v, cnt_v, sem):
    c = lax.axis_index("c")
    s = lax.axis_index("s")
    node0 = s * NODES_PER_TILE

    def zero_row(n, carry):
        for b in range(NUM_FEATS // LANES):
            rows_v[n, pl.ds(b * LANES, LANES)] = jnp.zeros((LANES,), jnp.float32)
        cnt_v[n, :] = jnp.zeros((LANES,), jnp.float32)
        return carry
    lax.fori_loop(0, NP_CHUNK, zero_row, 0)

    def zero_chunk(k, carry):
        r0 = node0 + k * NP_CHUNK
        pltpu.sync_copy(rows_v.at[pl.ds(0, NP_CHUNK)], acc_e.at[pl.ds(r0, NP_CHUNK)])
        pltpu.sync_copy(cnt_v.at[pl.ds(0, NP_CHUNK)], acc_c.at[pl.ds(r0, NP_CHUNK)])
        return carry
    lax.fori_loop(0, NP_STEPS, zero_chunk, 0)
    plsc.subcore_barrier()

    tile_base = c * EDGES_PER_CORE + s * EDGES_PER_TILE

    def chunk(i, carry):
        base = tile_base + i * CHUNK
        pltpu.sync_copy(src_ref.at[pl.ds(base, CHUNK)], idx_v)
        pltpu.sync_copy(dst_ref.at[pl.ds(base, CHUNK)], dst_v)
        pltpu.sync_copy(score_ref.at[pl.ds(base, CHUNK)], score_v)
        pltpu.sync_copy(cnt_ref.at[pl.ds(base, CHUNK)], cnt_v)
        pltpu.async_copy(emb_ref.at[idx_v], rows_v, sem).wait()

        def emul(e, inner):
            for b in range(NUM_FEATS // LANES):
                sl = pl.ds(b * LANES, LANES)
                rows_v[e, sl] = rows_v[e, sl] * score_v[e, sl]
            return inner
        lax.fori_loop(0, CHUNK, emul, 0)

        pltpu.sync_copy(rows_v, acc_e.at[dst_v], add=True)
        pltpu.sync_copy(cnt_v, acc_c.at[dst_v], add=True)
        return carry
    lax.fori_loop(0, NUM_CHUNKS, chunk, 0)
    plsc.subcore_barrier()

    def out_chunk(k, carry):
        r0 = node0 + k * NP_CHUNK
        pltpu.sync_copy(acc_e.at[pl.ds(r0, NP_CHUNK)], rows_v.at[pl.ds(0, NP_CHUNK)])
        pltpu.sync_copy(rows_v.at[pl.ds(0, NP_CHUNK)], sums_ref.at[c, pl.ds(r0, NP_CHUNK)])
        pltpu.sync_copy(acc_c.at[pl.ds(r0, NP_CHUNK)], cnt_v.at[pl.ds(0, NP_CHUNK)])
        pltpu.sync_copy(cnt_v.at[pl.ds(0, NP_CHUNK)], csums_ref.at[c, pl.ds(r0, NP_CHUNK)])
        return carry
    lax.fori_loop(0, NP_STEPS, out_chunk, 0)


@functools.partial(
    pl.kernel,
    out_type=(
        jax.ShapeDtypeStruct((NUM_CORES, N_PAD, NUM_FEATS), jnp.float32),
        jax.ShapeDtypeStruct((NUM_CORES, N_PAD, NUM_COUNTS), jnp.float32),
    ),
    mesh=plsc.VectorSubcoreMesh(
        core_axis_name="c", subcore_axis_name="s",
        num_cores=NUM_CORES, num_subcores=NUM_SUBCORES),
    scratch_types=[
        pltpu.VMEM_SHARED((N_PAD, NUM_FEATS), jnp.float32),
        pltpu.VMEM_SHARED((N_PAD, NUM_COUNTS), jnp.float32),
        pltpu.VMEM((CHUNK,), jnp.int32),
        pltpu.VMEM((CHUNK,), jnp.int32),
        pltpu.VMEM((CHUNK, NUM_FEATS), jnp.float32),
        pltpu.VMEM((CHUNK, NUM_FEATS), jnp.float32),
        pltpu.VMEM((CHUNK, NUM_COUNTS), jnp.float32),
        pltpu.SemaphoreType.DMA,
    ],
    compiler_params=pltpu.CompilerParams(use_tc_tiling_on_sc=False),
)
def _sc_aggregate(emb_ref, src_ref, dst_ref, score_ref, cnt_ref,
                  sums_ref, csums_ref, *scratch):
    _sc_body(emb_ref, src_ref, dst_ref, score_ref, cnt_ref,
             sums_ref, csums_ref, *scratch)


def _final_body(sums_ref, csums_ref, imp_ref, out_ref):
    imp = imp_ref[...]
    m = jnp.max(imp, axis=0, keepdims=True)
    e = jnp.exp(imp - m)
    w = e / jnp.sum(e, axis=0, keepdims=True)
    msg = sums_ref[0] + sums_ref[1]
    csum = csums_ref[0] + csums_ref[1]
    node_score = jnp.dot(csum, w, preferred_element_type=jnp.float32)
    out_ref[...] = msg / node_score


def _finalize(sums, csums, importance):
    return pl.pallas_call(
        _final_body,
        grid=(N_PAD // FIN_BLOCK,),
        in_specs=[
            pl.BlockSpec((NUM_CORES, FIN_BLOCK, NUM_FEATS), lambda i: (0, i, 0)),
            pl.BlockSpec((NUM_CORES, FIN_BLOCK, NUM_COUNTS), lambda i: (0, i, 0)),
            pl.BlockSpec((NUM_COUNTS, NUM_FEATS), lambda i: (0, 0)),
        ],
        out_specs=pl.BlockSpec((FIN_BLOCK, NUM_FEATS), lambda i: (i, 0)),
        out_shape=jax.ShapeDtypeStruct((N_PAD, NUM_FEATS), jnp.float32),
    )(sums, csums, importance)


def kernel(embedding, edge_index, cnt, importance):
    src = edge_index[0].astype(jnp.int32)
    dst = edge_index[1].astype(jnp.int32)
    score = _edge_scores(cnt, importance)
    sums, csums = _sc_aggregate(embedding, src, dst, score, cnt)
    out = _finalize(sums, csums, importance)
    return out[:N_NODES]


# speedup vs baseline: 4.0399x; 1.4000x over previous
import functools

import jax
import jax.numpy as jnp
from jax import lax
from jax.experimental import pallas as pl
from jax.experimental.pallas import tpu as pltpu
from jax.experimental.pallas import tpu_sc as plsc

N_NODES = 10000
N_EDGES = 320000
NUM_COUNTS = 16
NUM_FEATS = 128
LANES = 16

NUM_CORES = 2
NUM_SUBCORES = 16
EDGES_PER_CORE = N_EDGES // NUM_CORES
EDGES_PER_TILE = EDGES_PER_CORE // NUM_SUBCORES
CHUNK = 64
NCH = 156
TAIL = EDGES_PER_TILE - NCH * CHUNK
N_PAD = 10240
NODES_PER_TILE = N_PAD // NUM_SUBCORES
NP_CHUNK = 64
NP_STEPS = NODES_PER_TILE // NP_CHUNK

TC_BLOCK = 4000
FIN_BLOCK = 2048


def _score_body(cnt_ref, imp_ref, out_ref):
    imp = imp_ref[...]
    m = jnp.max(imp, axis=0, keepdims=True)
    e = jnp.exp(imp - m)
    w = e / jnp.sum(e, axis=0, keepdims=True)
    out_ref[...] = jnp.dot(cnt_ref[...], w, preferred_element_type=jnp.float32)


def _edge_scores(cnt, importance):
    return pl.pallas_call(
        _score_body,
        grid=(N_EDGES // TC_BLOCK,),
        in_specs=[
            pl.BlockSpec((TC_BLOCK, NUM_COUNTS), lambda i: (i, 0)),
            pl.BlockSpec((NUM_COUNTS, NUM_FEATS), lambda i: (0, 0)),
        ],
        out_specs=pl.BlockSpec((TC_BLOCK, NUM_FEATS), lambda i: (i, 0)),
        out_shape=jax.ShapeDtypeStruct((N_EDGES, NUM_FEATS), jnp.float32),
    )(cnt, importance)


def _sc_body(emb_ref, src_ref, dst_ref, score_ref, cnt_ref,
             sums_ref, csums_ref,
             acc_e, acc_c,
             idx0, dst0, score0, rows0, cnt0,
             idx1, dst1, score1, rows1, cnt1,
             idx_t, dst_t,
             sin0, sin1, sg0, sg1):
    c = lax.axis_index("c")
    s = lax.axis_index("s")
    node0 = s * NODES_PER_TILE
    tile_base = c * EDGES_PER_CORE + s * EDGES_PER_TILE

    bufs = ((idx0, dst0, score0, rows0, cnt0, sin0, sg0),
            (idx1, dst1, score1, rows1, cnt1, sin1, sg1))

    def zero_row(n, carry):
        for b in range(NUM_FEATS // LANES):
            rows0[n, pl.ds(b * LANES, LANES)] = jnp.zeros((LANES,), jnp.float32)
        cnt0[n, :] = jnp.zeros((LANES,), jnp.float32)
        return carry
    lax.fori_loop(0, NP_CHUNK, zero_row, 0)

    def zero_chunk(k, carry):
        r0 = node0 + k * NP_CHUNK
        pltpu.sync_copy(rows0.at[pl.ds(0, NP_CHUNK)], acc_e.at[pl.ds(r0, NP_CHUNK)])
        pltpu.sync_copy(cnt0.at[pl.ds(0, NP_CHUNK)], acc_c.at[pl.ds(r0, NP_CHUNK)])
        return carry
    lax.fori_loop(0, NP_STEPS, zero_chunk, 0)
    plsc.subcore_barrier()

    def in_pairs(i, b):
        idxb, dstb, scoreb, _, cntb, sinb, _ = bufs[b]
        base = tile_base + i * CHUNK
        return ((src_ref.at[pl.ds(base, CHUNK)], idxb),
                (dst_ref.at[pl.ds(base, CHUNK)], dstb),
                (score_ref.at[pl.ds(base, CHUNK)], scoreb),
                (cnt_ref.at[pl.ds(base, CHUNK)], cntb)), sinb

    def fire_in(i, b):
        pairs, sem = in_pairs(i, b)
        for src_, dst_ in pairs:
            pltpu.async_copy(src_, dst_, sem)

    def wait_in(i, b):
        pairs, sem = in_pairs(i, b)
        for src_, dst_ in pairs:
            pltpu.make_async_copy(src_, dst_, sem).wait()

    def fire_g(b):
        idxb, _, _, rowsb, _, _, sgb = bufs[b]
        pltpu.async_copy(emb_ref.at[idxb], rowsb, sgb)

    def wait_g(b):
        idxb, _, _, rowsb, _, _, sgb = bufs[b]
        pltpu.make_async_copy(emb_ref.at[idxb], rowsb, sgb).wait()

    def mul(b, nedges):
        _, _, scoreb, rowsb, _, _, _ = bufs[b]

        def em(k, carry):
            for eo in range(4):
                e = k * 4 + eo
                for b8 in range(NUM_FEATS // LANES):
                    sl = pl.ds(b8 * LANES, LANES)
                    rowsb[e, sl] = rowsb[e, sl] * scoreb[e, sl]
            return carry
        lax.fori_loop(0, nedges // 4, em, 0)

    def scatter(b):
        _, dstb, _, rowsb, cntb, _, _ = bufs[b]
        pltpu.sync_copy(rowsb, acc_e.at[dstb], add=True)
        pltpu.sync_copy(cntb, acc_c.at[dstb], add=True)

    fire_in(0, 0)
    wait_in(0, 0)
    fire_g(0)
    fire_in(1, 1)

    def pair_body(j, carry):
        i0 = 2 * j
        i1 = i0 + 1
        wait_in(i1, 1)
        wait_g(0)
        fire_g(1)
        mul(0, CHUNK)
        scatter(0)

        @pl.when(i0 + 2 < NCH)
        def _():
            fire_in(i0 + 2, 0)

        @pl.when(i1 + 1 < NCH)
        def _():
            wait_in(i1 + 1, 0)
            fire_g(0)

        wait_g(1)
        mul(1, CHUNK)
        scatter(1)

        @pl.when(i1 + 2 < NCH)
        def _():
            fire_in(i1 + 2, 1)
        return carry
    lax.fori_loop(0, NCH // 2, pair_body, 0)

    base_t = tile_base + NCH * CHUNK
    pltpu.sync_copy(src_ref.at[pl.ds(base_t, TAIL)], idx_t)
    pltpu.sync_copy(dst_ref.at[pl.ds(base_t, TAIL)], dst_t)
    pltpu.sync_copy(score_ref.at[pl.ds(base_t, TAIL)], score0.at[pl.ds(0, TAIL)])
    pltpu.sync_copy(cnt_ref.at[pl.ds(base_t, TAIL)], cnt0.at[pl.ds(0, TAIL)])
    pltpu.async_copy(emb_ref.at[idx_t], rows0.at[pl.ds(0, TAIL)], sg0).wait()
    mul(0, TAIL)
    pltpu.sync_copy(rows0.at[pl.ds(0, TAIL)], acc_e.at[dst_t], add=True)
    pltpu.sync_copy(cnt0.at[pl.ds(0, TAIL)], acc_c.at[dst_t], add=True)
    plsc.subcore_barrier()

    def out_chunk(k, carry):
        r0 = node0 + k * NP_CHUNK
        pltpu.sync_copy(acc_e.at[pl.ds(r0, NP_CHUNK)], rows0.at[pl.ds(0, NP_CHUNK)])
        pltpu.sync_copy(rows0.at[pl.ds(0, NP_CHUNK)], sums_ref.at[c, pl.ds(r0, NP_CHUNK)])
        pltpu.sync_copy(acc_c.at[pl.ds(r0, NP_CHUNK)], cnt0.at[pl.ds(0, NP_CHUNK)])
        pltpu.sync_copy(cnt0.at[pl.ds(0, NP_CHUNK)], csums_ref.at[c, pl.ds(r0, NP_CHUNK)])
        return carry
    lax.fori_loop(0, NP_STEPS, out_chunk, 0)


@functools.partial(
    pl.kernel,
    out_type=(
        jax.ShapeDtypeStruct((NUM_CORES, N_PAD, NUM_FEATS), jnp.float32),
        jax.ShapeDtypeStruct((NUM_CORES, N_PAD, NUM_COUNTS), jnp.float32),
    ),
    mesh=plsc.VectorSubcoreMesh(
        core_axis_name="c", subcore_axis_name="s",
        num_cores=NUM_CORES, num_subcores=NUM_SUBCORES),
    scratch_types=[
        pltpu.VMEM_SHARED((N_PAD, NUM_FEATS), jnp.float32),
        pltpu.VMEM_SHARED((N_PAD, NUM_COUNTS), jnp.float32),
        pltpu.VMEM((CHUNK,), jnp.int32),
        pltpu.VMEM((CHUNK,), jnp.int32),
        pltpu.VMEM((CHUNK, NUM_FEATS), jnp.float32),
        pltpu.VMEM((CHUNK, NUM_FEATS), jnp.float32),
        pltpu.VMEM((CHUNK, NUM_COUNTS), jnp.float32),
        pltpu.VMEM((CHUNK,), jnp.int32),
        pltpu.VMEM((CHUNK,), jnp.int32),
        pltpu.VMEM((CHUNK, NUM_FEATS), jnp.float32),
        pltpu.VMEM((CHUNK, NUM_FEATS), jnp.float32),
        pltpu.VMEM((CHUNK, NUM_COUNTS), jnp.float32),
        pltpu.VMEM((TAIL,), jnp.int32),
        pltpu.VMEM((TAIL,), jnp.int32),
        pltpu.SemaphoreType.DMA,
        pltpu.SemaphoreType.DMA,
        pltpu.SemaphoreType.DMA,
        pltpu.SemaphoreType.DMA,
    ],
    compiler_params=pltpu.CompilerParams(use_tc_tiling_on_sc=False),
)
def _sc_aggregate(emb_ref, src_ref, dst_ref, score_ref, cnt_ref,
                  sums_ref, csums_ref, *scratch):
    _sc_body(emb_ref, src_ref, dst_ref, score_ref, cnt_ref,
             sums_ref, csums_ref, *scratch)


def _final_body(sums_ref, csums_ref, imp_ref, out_ref):
    imp = imp_ref[...]
    m = jnp.max(imp, axis=0, keepdims=True)
    e = jnp.exp(imp - m)
    w = e / jnp.sum(e, axis=0, keepdims=True)
    msg = sums_ref[0] + sums_ref[1]
    csum = csums_ref[0] + csums_ref[1]
    node_score = jnp.dot(csum, w, preferred_element_type=jnp.float32)
    out_ref[...] = msg / node_score


def _finalize(sums, csums, importance):
    return pl.pallas_call(
        _final_body,
        grid=(N_PAD // FIN_BLOCK,),
        in_specs=[
            pl.BlockSpec((NUM_CORES, FIN_BLOCK, NUM_FEATS), lambda i: (0, i, 0)),
            pl.BlockSpec((NUM_CORES, FIN_BLOCK, NUM_COUNTS), lambda i: (0, i, 0)),
            pl.BlockSpec((NUM_COUNTS, NUM_FEATS), lambda i: (0, 0)),
        ],
        out_specs=pl.BlockSpec((FIN_BLOCK, NUM_FEATS), lambda i: (i, 0)),
        out_shape=jax.ShapeDtypeStruct((N_PAD, NUM_FEATS), jnp.float32),
    )(sums, csums, importance)


def kernel(embedding, edge_index, cnt, importance):
    src = edge_index[0].astype(jnp.int32)
    dst = edge_index[1].astype(jnp.int32)
    score = _edge_scores(cnt, importance)
    sums, csums = _sc_aggregate(embedding, src, dst, score, cnt)
    out = _finalize(sums, csums, importance)
    return out[:N_NODES]


# speedup vs baseline: 4.0841x; 1.0109x over previous
import functools

import jax
import jax.numpy as jnp
from jax import lax
from jax.experimental import pallas as pl
from jax.experimental.pallas import tpu as pltpu
from jax.experimental.pallas import tpu_sc as plsc

N_NODES = 10000
N_EDGES = 320000
NUM_COUNTS = 16
NUM_FEATS = 128
LANES = 16

NUM_SEGS = 2
E_SEG = N_EDGES // NUM_SEGS
NUM_CORES = 2
NUM_SUBCORES = 16
SEG_PER_CORE = E_SEG // NUM_CORES
SEG_PER_TILE = SEG_PER_CORE // NUM_SUBCORES
CHUNK = 64
NCH = SEG_PER_TILE // CHUNK
TAIL = SEG_PER_TILE - NCH * CHUNK
N_PAD = 10240
NODES_PER_TILE = N_PAD // NUM_SUBCORES
NP_CHUNK = 64
NP_STEPS = NODES_PER_TILE // NP_CHUNK

TC_BLOCK = 4000
FIN_BLOCK = 2048


def _score_body(cnt_ref, imp_ref, out_ref):
    imp = imp_ref[...]
    m = jnp.max(imp, axis=0, keepdims=True)
    e = jnp.exp(imp - m)
    w = e / jnp.sum(e, axis=0, keepdims=True)
    out_ref[...] = jnp.dot(cnt_ref[...], w, preferred_element_type=jnp.float32)


def _edge_scores(cnt, importance, seg):
    nblk = E_SEG // TC_BLOCK
    return pl.pallas_call(
        _score_body,
        grid=(nblk,),
        in_specs=[
            pl.BlockSpec((TC_BLOCK, NUM_COUNTS), lambda i: (i + seg * nblk, 0)),
            pl.BlockSpec((NUM_COUNTS, NUM_FEATS), lambda i: (0, 0)),
        ],
        out_specs=pl.BlockSpec((TC_BLOCK, NUM_FEATS), lambda i: (i, 0)),
        out_shape=jax.ShapeDtypeStruct((E_SEG, NUM_FEATS), jnp.float32),
    )(cnt, importance)


def _sc_body(seg, emb_ref, src_ref, dst_ref, score_ref, cnt_ref,
             sums_ref, csums_ref,
             acc_e, acc_c,
             idx0, dst0, score0, rows0, cnt0,
             idx1, dst1, score1, rows1, cnt1,
             idx_t, dst_t,
             sin0, sin1, sg0, sg1):
    c = lax.axis_index("c")
    s = lax.axis_index("s")
    node0 = s * NODES_PER_TILE
    loc_base = c * SEG_PER_CORE + s * SEG_PER_TILE
    glob_base = seg * E_SEG + loc_base

    bufs = ((idx0, dst0, score0, rows0, cnt0, sin0, sg0),
            (idx1, dst1, score1, rows1, cnt1, sin1, sg1))

    def zero_row(n, carry):
        for b in range(NUM_FEATS // LANES):
            rows0[n, pl.ds(b * LANES, LANES)] = jnp.zeros((LANES,), jnp.float32)
        cnt0[n, :] = jnp.zeros((LANES,), jnp.float32)
        return carry
    lax.fori_loop(0, NP_CHUNK, zero_row, 0)

    def zero_chunk(k, carry):
        r0 = node0 + k * NP_CHUNK
        pltpu.sync_copy(rows0.at[pl.ds(0, NP_CHUNK)], acc_e.at[pl.ds(r0, NP_CHUNK)])
        pltpu.sync_copy(cnt0.at[pl.ds(0, NP_CHUNK)], acc_c.at[pl.ds(r0, NP_CHUNK)])
        return carry
    lax.fori_loop(0, NP_STEPS, zero_chunk, 0)
    plsc.subcore_barrier()

    def in_pairs(i, b):
        idxb, dstb, scoreb, _, cntb, sinb, _ = bufs[b]
        gbase = glob_base + i * CHUNK
        lbase = loc_base + i * CHUNK
        return ((src_ref.at[pl.ds(gbase, CHUNK)], idxb),
                (dst_ref.at[pl.ds(gbase, CHUNK)], dstb),
                (score_ref.at[pl.ds(lbase, CHUNK)], scoreb),
                (cnt_ref.at[pl.ds(gbase, CHUNK)], cntb)), sinb

    def fire_in(i, b):
        pairs, sem = in_pairs(i, b)
        for src_, dst_ in pairs:
            pltpu.async_copy(src_, dst_, sem)

    def wait_in(i, b):
        pairs, sem = in_pairs(i, b)
        for src_, dst_ in pairs:
            pltpu.make_async_copy(src_, dst_, sem).wait()

    def fire_g(b):
        idxb, _, _, rowsb, _, _, sgb = bufs[b]
        pltpu.async_copy(emb_ref.at[idxb], rowsb, sgb)

    def wait_g(b):
        idxb, _, _, rowsb, _, _, sgb = bufs[b]
        pltpu.make_async_copy(emb_ref.at[idxb], rowsb, sgb).wait()

    def mul(b, nedges):
        _, _, scoreb, rowsb, _, _, _ = bufs[b]

        def em(k, carry):
            for eo in range(4):
                e = k * 4 + eo
                for b8 in range(NUM_FEATS // LANES):
                    sl = pl.ds(b8 * LANES, LANES)
                    rowsb[e, sl] = rowsb[e, sl] * scoreb[e, sl]
            return carry
        lax.fori_loop(0, nedges // 4, em, 0)

    def scatter(b):
        _, dstb, _, rowsb, cntb, _, _ = bufs[b]
        pltpu.sync_copy(rowsb, acc_e.at[dstb], add=True)
        pltpu.sync_copy(cntb, acc_c.at[dstb], add=True)

    fire_in(0, 0)
    wait_in(0, 0)
    fire_g(0)
    fire_in(1, 1)

    def pair_body(j, carry):
        i0 = 2 * j
        i1 = i0 + 1
        wait_in(i1, 1)
        wait_g(0)
        fire_g(1)
        mul(0, CHUNK)
        scatter(0)

        @pl.when(i0 + 2 < NCH)
        def _():
            fire_in(i0 + 2, 0)

        @pl.when(i1 + 1 < NCH)
        def _():
            wait_in(i1 + 1, 0)
            fire_g(0)

        wait_g(1)
        mul(1, CHUNK)
        scatter(1)

        @pl.when(i1 + 2 < NCH)
        def _():
            fire_in(i1 + 2, 1)
        return carry
    lax.fori_loop(0, NCH // 2, pair_body, 0)

    gbase_t = glob_base + NCH * CHUNK
    lbase_t = loc_base + NCH * CHUNK
    pltpu.sync_copy(src_ref.at[pl.ds(gbase_t, TAIL)], idx_t)
    pltpu.sync_copy(dst_ref.at[pl.ds(gbase_t, TAIL)], dst_t)
    pltpu.sync_copy(score_ref.at[pl.ds(lbase_t, TAIL)], score0.at[pl.ds(0, TAIL)])
    pltpu.sync_copy(cnt_ref.at[pl.ds(gbase_t, TAIL)], cnt0.at[pl.ds(0, TAIL)])
    pltpu.async_copy(emb_ref.at[idx_t], rows0.at[pl.ds(0, TAIL)], sg0).wait()
    mul(0, TAIL)
    pltpu.sync_copy(rows0.at[pl.ds(0, TAIL)], acc_e.at[dst_t], add=True)
    pltpu.sync_copy(cnt0.at[pl.ds(0, TAIL)], acc_c.at[dst_t], add=True)
    plsc.subcore_barrier()

    def out_chunk(k, carry):
        r0 = node0 + k * NP_CHUNK
        pltpu.sync_copy(acc_e.at[pl.ds(r0, NP_CHUNK)], rows0.at[pl.ds(0, NP_CHUNK)])
        pltpu.sync_copy(rows0.at[pl.ds(0, NP_CHUNK)], sums_ref.at[c, pl.ds(r0, NP_CHUNK)])
        pltpu.sync_copy(acc_c.at[pl.ds(r0, NP_CHUNK)], cnt0.at[pl.ds(0, NP_CHUNK)])
        pltpu.sync_copy(cnt0.at[pl.ds(0, NP_CHUNK)], csums_ref.at[c, pl.ds(r0, NP_CHUNK)])
        return carry
    lax.fori_loop(0, NP_STEPS, out_chunk, 0)


def _make_sc_aggregate(seg):
    @functools.partial(
        pl.kernel,
        out_type=(
            jax.ShapeDtypeStruct((NUM_CORES, N_PAD, NUM_FEATS), jnp.float32),
            jax.ShapeDtypeStruct((NUM_CORES, N_PAD, NUM_COUNTS), jnp.float32),
        ),
        mesh=plsc.VectorSubcoreMesh(
            core_axis_name="c", subcore_axis_name="s",
            num_cores=NUM_CORES, num_subcores=NUM_SUBCORES),
        scratch_types=[
            pltpu.VMEM_SHARED((N_PAD, NUM_FEATS), jnp.float32),
            pltpu.VMEM_SHARED((N_PAD, NUM_COUNTS), jnp.float32),
            pltpu.VMEM((CHUNK,), jnp.int32),
            pltpu.VMEM((CHUNK,), jnp.int32),
            pltpu.VMEM((CHUNK, NUM_FEATS), jnp.float32),
            pltpu.VMEM((CHUNK, NUM_FEATS), jnp.float32),
            pltpu.VMEM((CHUNK, NUM_COUNTS), jnp.float32),
            pltpu.VMEM((CHUNK,), jnp.int32),
            pltpu.VMEM((CHUNK,), jnp.int32),
            pltpu.VMEM((CHUNK, NUM_FEATS), jnp.float32),
            pltpu.VMEM((CHUNK, NUM_FEATS), jnp.float32),
            pltpu.VMEM((CHUNK, NUM_COUNTS), jnp.float32),
            pltpu.VMEM((TAIL,), jnp.int32),
            pltpu.VMEM((TAIL,), jnp.int32),
            pltpu.SemaphoreType.DMA,
            pltpu.SemaphoreType.DMA,
            pltpu.SemaphoreType.DMA,
            pltpu.SemaphoreType.DMA,
        ],
        compiler_params=pltpu.CompilerParams(use_tc_tiling_on_sc=False),
    )
    def _sc(emb_ref, src_ref, dst_ref, score_ref, cnt_ref,
            sums_ref, csums_ref, *scratch):
        _sc_body(seg, emb_ref, src_ref, dst_ref, score_ref, cnt_ref,
                 sums_ref, csums_ref, *scratch)
    return _sc


_SC_SEG = tuple(_make_sc_aggregate(seg) for seg in range(NUM_SEGS))


def _final_body(sa_ref, sb_ref, ca_ref, cb_ref, imp_ref, out_ref):
    imp = imp_ref[...]
    m = jnp.max(imp, axis=0, keepdims=True)
    e = jnp.exp(imp - m)
    w = e / jnp.sum(e, axis=0, keepdims=True)
    msg = sa_ref[0] + sa_ref[1] + sb_ref[0] + sb_ref[1]
    csum = ca_ref[0] + ca_ref[1] + cb_ref[0] + cb_ref[1]
    node_score = jnp.dot(csum, w, preferred_element_type=jnp.float32)
    out_ref[...] = msg / node_score


def _finalize(sums0, sums1, csums0, csums1, importance):
    return pl.pallas_call(
        _final_body,
        grid=(N_PAD // FIN_BLOCK,),
        in_specs=[
            pl.BlockSpec((NUM_CORES, FIN_BLOCK, NUM_FEATS), lambda i: (0, i, 0)),
            pl.BlockSpec((NUM_CORES, FIN_BLOCK, NUM_FEATS), lambda i: (0, i, 0)),
            pl.BlockSpec((NUM_CORES, FIN_BLOCK, NUM_COUNTS), lambda i: (0, i, 0)),
            pl.BlockSpec((NUM_CORES, FIN_BLOCK, NUM_COUNTS), lambda i: (0, i, 0)),
            pl.BlockSpec((NUM_COUNTS, NUM_FEATS), lambda i: (0, 0)),
        ],
        out_specs=pl.BlockSpec((FIN_BLOCK, NUM_FEATS), lambda i: (i, 0)),
        out_shape=jax.ShapeDtypeStruct((N_PAD, NUM_FEATS), jnp.float32),
    )(sums0, sums1, csums0, csums1, importance)


def kernel(embedding, edge_index, cnt, importance):
    src = edge_index[0].astype(jnp.int32)
    dst = edge_index[1].astype(jnp.int32)
    score0 = _edge_scores(cnt, importance, 0)
    score1 = _edge_scores(cnt, importance, 1)
    sums0, csums0 = _SC_SEG[0](embedding, src, dst, score0, cnt)
    sums1, csums1 = _SC_SEG[1](embedding, src, dst, score1, cnt)
    out = _finalize(sums0, sums1, csums0, csums1, importance)
    return out[:N_NODES]


# speedup vs baseline: 4.0871x; 1.0007x over previous
import functools

import jax
import jax.numpy as jnp
from jax import lax
from jax.experimental import pallas as pl
from jax.experimental.pallas import tpu as pltpu
from jax.experimental.pallas import tpu_sc as plsc

N_NODES = 10000
N_EDGES = 320000
NUM_COUNTS = 16
NUM_FEATS = 128
LANES = 16

NUM_SEGS = 2
E_SEG = N_EDGES // NUM_SEGS
NUM_CORES = 2
NUM_SUBCORES = 16
SEG_PER_CORE = E_SEG // NUM_CORES
SEG_PER_TILE = SEG_PER_CORE // NUM_SUBCORES
CHUNK = 64
NCH = SEG_PER_TILE // CHUNK
TAIL = SEG_PER_TILE - NCH * CHUNK
N_PAD = 10240
NODES_PER_TILE = N_PAD // NUM_SUBCORES
NP_CHUNK = 64
NP_STEPS = NODES_PER_TILE // NP_CHUNK

TC_BLOCK = 4000
FIN_BLOCK = 2048


def _score_body(cnt_ref, imp_ref, out_ref):
    imp = imp_ref[...]
    m = jnp.max(imp, axis=0, keepdims=True)
    e = jnp.exp(imp - m)
    w = e / jnp.sum(e, axis=0, keepdims=True)
    out_ref[...] = jnp.dot(cnt_ref[...], w, preferred_element_type=jnp.float32)


def _edge_scores(cnt, importance, seg):
    nblk = E_SEG // TC_BLOCK
    return pl.pallas_call(
        _score_body,
        grid=(nblk,),
        in_specs=[
            pl.BlockSpec((TC_BLOCK, NUM_COUNTS), lambda i: (i + seg * nblk, 0)),
            pl.BlockSpec((NUM_COUNTS, NUM_FEATS), lambda i: (0, 0)),
        ],
        out_specs=pl.BlockSpec((TC_BLOCK, NUM_FEATS), lambda i: (i, 0)),
        out_shape=jax.ShapeDtypeStruct((E_SEG, NUM_FEATS), jnp.float32),
    )(cnt, importance)


def _sc_body(seg, emb_ref, src_ref, dst_ref, score_ref, cnt_ref,
             sums_ref, csums_ref,
             acc_e, acc_c,
             idx0, dst0, score0, rows0, cnt0,
             idx1, dst1, score1, rows1, cnt1,
             idx_t, dst_t,
             sin0, sin1, sg0, sg1):
    c = lax.axis_index("c")
    s = lax.axis_index("s")
    node0 = s * NODES_PER_TILE
    loc_base = c * SEG_PER_CORE + s * SEG_PER_TILE
    glob_base = seg * E_SEG + loc_base

    bufs = ((idx0, dst0, score0, rows0, cnt0, sin0, sg0),
            (idx1, dst1, score1, rows1, cnt1, sin1, sg1))

    def zero_row(n, carry):
        for b in range(NUM_FEATS // LANES):
            rows0[n, pl.ds(b * LANES, LANES)] = jnp.zeros((LANES,), jnp.float32)
        cnt0[n, :] = jnp.zeros((LANES,), jnp.float32)
        return carry
    lax.fori_loop(0, NP_CHUNK, zero_row, 0)

    def zero_chunk(k, carry):
        r0 = node0 + k * NP_CHUNK
        pltpu.sync_copy(rows0.at[pl.ds(0, NP_CHUNK)], acc_e.at[pl.ds(r0, NP_CHUNK)])
        pltpu.sync_copy(cnt0.at[pl.ds(0, NP_CHUNK)], acc_c.at[pl.ds(r0, NP_CHUNK)])
        return carry
    lax.fori_loop(0, NP_STEPS, zero_chunk, 0)
    plsc.subcore_barrier()

    def in_pairs(i, b):
        idxb, dstb, scoreb, _, cntb, sinb, _ = bufs[b]
        gbase = glob_base + i * CHUNK
        lbase = loc_base + i * CHUNK
        return ((src_ref.at[pl.ds(gbase, CHUNK)], idxb),
                (dst_ref.at[pl.ds(gbase, CHUNK)], dstb),
                (score_ref.at[pl.ds(lbase, CHUNK)], scoreb),
                (cnt_ref.at[pl.ds(gbase, CHUNK)], cntb)), sinb

    def fire_in(i, b):
        pairs, sem = in_pairs(i, b)
        for src_, dst_ in pairs:
            pltpu.async_copy(src_, dst_, sem)

    def wait_in(i, b):
        pairs, sem = in_pairs(i, b)
        for src_, dst_ in pairs:
            pltpu.make_async_copy(src_, dst_, sem).wait()

    def fire_g(b):
        idxb, _, _, rowsb, _, _, sgb = bufs[b]
        pltpu.async_copy(emb_ref.at[idxb], rowsb, sgb)

    def wait_g(b):
        idxb, _, _, rowsb, _, _, sgb = bufs[b]
        pltpu.make_async_copy(emb_ref.at[idxb], rowsb, sgb).wait()

    def mul(b, nedges):
        _, _, scoreb, rowsb, _, _, _ = bufs[b]

        @plsc.parallel_loop(0, nedges, step=1, unroll=4)
        def _(e):
            for b8 in range(NUM_FEATS // LANES):
                sl = pl.ds(b8 * LANES, LANES)
                rowsb[e, sl] = rowsb[e, sl] * scoreb[e, sl]

    def scatter(b):
        _, dstb, _, rowsb, cntb, _, _ = bufs[b]
        pltpu.sync_copy(rowsb, acc_e.at[dstb], add=True)
        pltpu.sync_copy(cntb, acc_c.at[dstb], add=True)

    fire_in(0, 0)
    wait_in(0, 0)
    fire_g(0)
    fire_in(1, 1)

    def pair_body(j, carry):
        i0 = 2 * j
        i1 = i0 + 1
        wait_in(i1, 1)
        wait_g(0)
        fire_g(1)
        mul(0, CHUNK)
        scatter(0)

        @pl.when(i0 + 2 < NCH)
        def _():
            fire_in(i0 + 2, 0)

        @pl.when(i1 + 1 < NCH)
        def _():
            wait_in(i1 + 1, 0)
            fire_g(0)

        wait_g(1)
        mul(1, CHUNK)
        scatter(1)

        @pl.when(i1 + 2 < NCH)
        def _():
            fire_in(i1 + 2, 1)
        return carry
    lax.fori_loop(0, NCH // 2, pair_body, 0)

    gbase_t = glob_base + NCH * CHUNK
    lbase_t = loc_base + NCH * CHUNK
    pltpu.sync_copy(src_ref.at[pl.ds(gbase_t, TAIL)], idx_t)
    pltpu.sync_copy(dst_ref.at[pl.ds(gbase_t, TAIL)], dst_t)
    pltpu.sync_copy(score_ref.at[pl.ds(lbase_t, TAIL)], score0.at[pl.ds(0, TAIL)])
    pltpu.sync_copy(cnt_ref.at[pl.ds(gbase_t, TAIL)], cnt0.at[pl.ds(0, TAIL)])
    pltpu.async_copy(emb_ref.at[idx_t], rows0.at[pl.ds(0, TAIL)], sg0).wait()
    mul(0, TAIL)
    pltpu.sync_copy(rows0.at[pl.ds(0, TAIL)], acc_e.at[dst_t], add=True)
    pltpu.sync_copy(cnt0.at[pl.ds(0, TAIL)], acc_c.at[dst_t], add=True)
    plsc.subcore_barrier()

    def out_chunk(k, carry):
        r0 = node0 + k * NP_CHUNK
        pltpu.sync_copy(acc_e.at[pl.ds(r0, NP_CHUNK)], rows0.at[pl.ds(0, NP_CHUNK)])
        pltpu.sync_copy(rows0.at[pl.ds(0, NP_CHUNK)], sums_ref.at[c, pl.ds(r0, NP_CHUNK)])
        pltpu.sync_copy(acc_c.at[pl.ds(r0, NP_CHUNK)], cnt0.at[pl.ds(0, NP_CHUNK)])
        pltpu.sync_copy(cnt0.at[pl.ds(0, NP_CHUNK)], csums_ref.at[c, pl.ds(r0, NP_CHUNK)])
        return carry
    lax.fori_loop(0, NP_STEPS, out_chunk, 0)


def _make_sc_aggregate(seg):
    @functools.partial(
        pl.kernel,
        out_type=(
            jax.ShapeDtypeStruct((NUM_CORES, N_PAD, NUM_FEATS), jnp.float32),
            jax.ShapeDtypeStruct((NUM_CORES, N_PAD, NUM_COUNTS), jnp.float32),
        ),
        mesh=plsc.VectorSubcoreMesh(
            core_axis_name="c", subcore_axis_name="s",
            num_cores=NUM_CORES, num_subcores=NUM_SUBCORES),
        scratch_types=[
            pltpu.VMEM_SHARED((N_PAD, NUM_FEATS), jnp.float32),
            pltpu.VMEM_SHARED((N_PAD, NUM_COUNTS), jnp.float32),
            pltpu.VMEM((CHUNK,), jnp.int32),
            pltpu.VMEM((CHUNK,), jnp.int32),
            pltpu.VMEM((CHUNK, NUM_FEATS), jnp.float32),
            pltpu.VMEM((CHUNK, NUM_FEATS), jnp.float32),
            pltpu.VMEM((CHUNK, NUM_COUNTS), jnp.float32),
            pltpu.VMEM((CHUNK,), jnp.int32),
            pltpu.VMEM((CHUNK,), jnp.int32),
            pltpu.VMEM((CHUNK, NUM_FEATS), jnp.float32),
            pltpu.VMEM((CHUNK, NUM_FEATS), jnp.float32),
            pltpu.VMEM((CHUNK, NUM_COUNTS), jnp.float32),
            pltpu.VMEM((TAIL,), jnp.int32),
            pltpu.VMEM((TAIL,), jnp.int32),
            pltpu.SemaphoreType.DMA,
            pltpu.SemaphoreType.DMA,
            pltpu.SemaphoreType.DMA,
            pltpu.SemaphoreType.DMA,
        ],
        compiler_params=pltpu.CompilerParams(use_tc_tiling_on_sc=False),
    )
    def _sc(emb_ref, src_ref, dst_ref, score_ref, cnt_ref,
            sums_ref, csums_ref, *scratch):
        _sc_body(seg, emb_ref, src_ref, dst_ref, score_ref, cnt_ref,
                 sums_ref, csums_ref, *scratch)
    return _sc


_SC_SEG = tuple(_make_sc_aggregate(seg) for seg in range(NUM_SEGS))


def _final_body(sa_ref, sb_ref, ca_ref, cb_ref, imp_ref, out_ref):
    imp = imp_ref[...]
    m = jnp.max(imp, axis=0, keepdims=True)
    e = jnp.exp(imp - m)
    w = e / jnp.sum(e, axis=0, keepdims=True)
    msg = sa_ref[0] + sa_ref[1] + sb_ref[0] + sb_ref[1]
    csum = ca_ref[0] + ca_ref[1] + cb_ref[0] + cb_ref[1]
    node_score = jnp.dot(csum, w, preferred_element_type=jnp.float32)
    out_ref[...] = msg / node_score


def _finalize(sums0, sums1, csums0, csums1, importance):
    return pl.pallas_call(
        _final_body,
        grid=(N_PAD // FIN_BLOCK,),
        in_specs=[
            pl.BlockSpec((NUM_CORES, FIN_BLOCK, NUM_FEATS), lambda i: (0, i, 0)),
            pl.BlockSpec((NUM_CORES, FIN_BLOCK, NUM_FEATS), lambda i: (0, i, 0)),
            pl.BlockSpec((NUM_CORES, FIN_BLOCK, NUM_COUNTS), lambda i: (0, i, 0)),
            pl.BlockSpec((NUM_CORES, FIN_BLOCK, NUM_COUNTS), lambda i: (0, i, 0)),
            pl.BlockSpec((NUM_COUNTS, NUM_FEATS), lambda i: (0, 0)),
        ],
        out_specs=pl.BlockSpec((FIN_BLOCK, NUM_FEATS), lambda i: (i, 0)),
        out_shape=jax.ShapeDtypeStruct((N_PAD, NUM_FEATS), jnp.float32),
    )(sums0, sums1, csums0, csums1, importance)


def kernel(embedding, edge_index, cnt, importance):
    src = edge_index[0].astype(jnp.int32)
    dst = edge_index[1].astype(jnp.int32)
    score0 = _edge_scores(cnt, importance, 0)
    score1 = _edge_scores(cnt, importance, 1)
    sums0, csums0 = _SC_SEG[0](embedding, src, dst, score0, cnt)
    sums1, csums1 = _SC_SEG[1](embedding, src, dst, score1, cnt)
    out = _finalize(sums0, sums1, csums0, csums1, importance)
    return out[:N_NODES]
